# baseline (speedup 1.0000x reference)
import jax
import jax.numpy as jnp
from jax import lax
from jax.experimental import pallas as pl
from jax.experimental.pallas import tpu as pltpu
from jax.experimental.pallas import tpu_sc as plsc

_NC = 2
_NS = 16
_NW = _NC * _NS

_D = 1024
_BATCH = 4
_SEQ = 8192
_B = _BATCH * _SEQ
_BPW = _B // _NW
_WPB = _SEQ // _BPW
_C = 56
_SIZES = [_C] * (_BPW // _C) + ([_BPW % _C] if _BPW % _C else [])
_OFFS = [i * _C for i in range(len(_SIZES))]
_NCHUNK = len(_SIZES)
_NBUF = 2


def _gather_body(table_hbm, idx_hbm, out_hbm, idx_v, buf, gsem, psem):
    wid = lax.axis_index("s") * _NC + lax.axis_index("c")
    b = wid // _WPB
    s0 = (wid % _WPB) * _BPW
    pltpu.sync_copy(idx_hbm.at[b, pl.ds(s0, _BPW)], idx_v)

    def gather(j):
        return pltpu.make_async_copy(
            table_hbm.at[idx_v.at[pl.ds(_OFFS[j], _SIZES[j])]],
            buf.at[j % _NBUF, pl.ds(0, _SIZES[j])],
            gsem.at[j % _NBUF])

    def put(j):
        return pltpu.make_async_copy(
            buf.at[j % _NBUF, pl.ds(0, _SIZES[j])],
            out_hbm.at[b, pl.ds(s0 + _OFFS[j], _SIZES[j])],
            psem.at[j % _NBUF])

    for j in range(_NBUF - 1):
        gather(j).start()
    for j in range(_NCHUNK):
        gather(j).wait()
        put(j).start()
        nxt = j + _NBUF - 1
        if nxt < _NCHUNK:
            if j >= 1:
                put(j - 1).wait()
            gather(nxt).start()
    for j in range(_NCHUNK - _NBUF, _NCHUNK):
        put(j).wait()


def kernel(encodes, X):
    mesh = plsc.VectorSubcoreMesh(core_axis_name="c", subcore_axis_name="s")
    fn = pl.kernel(
        _gather_body,
        out_type=jax.ShapeDtypeStruct((_BATCH, _SEQ, _D), jnp.float32),
        mesh=mesh,
        scratch_types=[
            pltpu.VMEM((_BPW,), jnp.int32),
            pltpu.VMEM((_NBUF, _C, _D), jnp.float32),
            pltpu.SemaphoreType.DMA((_NBUF,)),
            pltpu.SemaphoreType.DMA((_NBUF,)),
        ],
    )
    return fn(encodes, X.astype(jnp.int32))


# speedup vs baseline: 1.0256x; 1.0256x over previous
import jax
import jax.numpy as jnp
from jax import lax
from jax.experimental import pallas as pl
from jax.experimental.pallas import tpu as pltpu
from jax.experimental.pallas import tpu_sc as plsc

_NC = 2
_NS = 16
_NW = _NC * _NS

_D = 1024
_BATCH = 4
_SEQ = 8192
_B = _BATCH * _SEQ
_BPW = _B // _NW
_WPB = _SEQ // _BPW
_C = 32
_NCHUNK = _BPW // _C
_NBUF = 3


def _gather_body(table_hbm, idx_hbm, out_hbm, idx_v, buf, gsem, psem):
    wid = lax.axis_index("s") * _NC + lax.axis_index("c")
    b = wid // _WPB
    s0 = (wid % _WPB) * _BPW
    pltpu.sync_copy(idx_hbm.at[b, pl.ds(s0, _BPW)], idx_v)

    def gather(j):
        return pltpu.make_async_copy(
            table_hbm.at[idx_v.at[pl.ds(j * _C, _C)]], buf.at[j % _NBUF],
            gsem.at[j % _NBUF])

    def put(j):
        return pltpu.make_async_copy(
            buf.at[j % _NBUF], out_hbm.at[b, pl.ds(s0 + j * _C, _C)],
            psem.at[j % _NBUF])

    for j in range(_NBUF - 1):
        gather(j).start()
    for j in range(_NCHUNK):
        gather(j).wait()
        put(j).start()
        nxt = j + _NBUF - 1
        if nxt < _NCHUNK:
            if j >= 1:
                put(j - 1).wait()
            gather(nxt).start()
    for j in range(_NCHUNK - _NBUF, _NCHUNK):
        put(j).wait()


def kernel(encodes, X):
    mesh = plsc.VectorSubcoreMesh(core_axis_name="c", subcore_axis_name="s")
    fn = pl.kernel(
        _gather_body,
        out_type=jax.ShapeDtypeStruct((_BATCH, _SEQ, _D), jnp.float32),
        mesh=mesh,
        scratch_types=[
            pltpu.VMEM((_BPW,), jnp.int32),
            pltpu.VMEM((_NBUF, _C, _D), jnp.float32),
            pltpu.SemaphoreType.DMA((_NBUF,)),
            pltpu.SemaphoreType.DMA((_NBUF,)),
        ],
    )
    return fn(encodes, X.astype(jnp.int32))
